# baseline (speedup 1.0000x reference)
import functools

import jax
import jax.numpy as jnp
from jax import lax
from jax.experimental import pallas as pl
from jax.experimental.pallas import tpu as pltpu
from jax.experimental.pallas import tpu_sc as plsc

N = 10000
E = 320000
F = 128
FO = 256

N_PAD = 10240
E_AUG = E + N
E_PAD = 360448
EROWS = E_PAD // 128
CH_AGG = EROWS // 32
CH_DEG = EROWS // 16
GRP = 8
NSLICE = N_PAD // 16


def _sc_aggregate(row2d, col2d, w2d, x_pad):
    mesh = plsc.VectorSubcoreMesh(core_axis_name="c", subcore_axis_name="s")

    @functools.partial(
        pl.kernel,
        out_type=jax.ShapeDtypeStruct((2, N_PAD, F), jnp.float32),
        mesh=mesh,
        compiler_params=pltpu.CompilerParams(needs_layout_passes=False),
        scratch_types=[
            pltpu.VMEM((GRP, 128), jnp.int32),
            pltpu.VMEM((GRP, 128), jnp.int32),
            pltpu.VMEM((GRP, 128), jnp.float32),
            pltpu.VMEM((N_PAD,), jnp.float32),
            pltpu.VMEM((2, 128, F), jnp.float32),
            pltpu.VMEM((NSLICE,), jnp.float32),
            pltpu.SemaphoreType.DMA,
            pltpu.SemaphoreType.DMA,
            pltpu.SemaphoreType.DMA,
            pltpu.SemaphoreType.DMA,
            pltpu.SemaphoreType.DMA,
            pltpu.VMEM_SHARED((N_PAD,), jnp.float32),
            pltpu.VMEM_SHARED((N_PAD, F), jnp.float32),
        ],
    )
    def k(row_hbm, col_hbm, w_hbm, x_hbm, out_hbm,
          rowg, colg, nrmg, dis_v, xbuf, zbuf, sem, sg0, sg1, ss0, ss1,
          sp_deg, sp_agg):
        sem_g = (sg0, sg1)
        sem_s = (ss0, ss1)
        cid = lax.axis_index("c")
        sid = lax.axis_index("s")
        wid = sid * 2 + cid
        nbase = sid * NSLICE
        z16 = jnp.zeros((16,), jnp.float32)
        nsl = pl.ds(nbase, NSLICE)

        def zero_xrow(i, c):
            for kk in range(8):
                xbuf[0, i, pl.ds(kk * 16, 16)] = z16
            return c
        lax.fori_loop(0, 128, zero_xrow, 0)
        for i in range(NSLICE // 16):
            zbuf[pl.ds(i * 16, 16)] = z16
        pltpu.sync_copy(zbuf, sp_deg.at[nsl])
        for j in range(NSLICE // 128):
            pltpu.sync_copy(xbuf.at[0], sp_agg.at[pl.ds(nbase + j * 128, 128)])
        plsc.subcore_barrier()

        scope_deg = jax.named_scope("ph_deg")
        scope_deg.__enter__()

        def deg_grp(gp, c):
            base = sid * CH_DEG + gp * GRP
            pltpu.sync_copy(col_hbm.at[pl.ds(base, GRP)], colg)
            pltpu.sync_copy(w_hbm.at[pl.ds(base, GRP)], nrmg)
            descs = [pltpu.async_copy(nrmg.at[j], sp_deg.at[colg.at[j]], sem,
                                      add=True)
                     for j in range(GRP)]
            for d in descs:
                d.wait()
            return c
        lax.fori_loop(0, CH_DEG // GRP, deg_grp, 0)
        plsc.subcore_barrier()
        scope_deg.__exit__(None, None, None)

        pltpu.sync_copy(sp_deg.at[nsl], zbuf)

        def dis_step(i, c):
            d = zbuf[pl.ds(i * 16, 16)]
            bits = lax.bitcast_convert_type(d, jnp.int32)
            bits = jnp.int32(0x5F3759DF) - jnp.right_shift(bits, jnp.int32(1))
            y = lax.bitcast_convert_type(bits, jnp.float32)
            half = -0.5 * d
            for _ in range(3):
                y = y * (1.5 + half * y * y)
            zbuf[pl.ds(i * 16, 16)] = y
            return c
        lax.fori_loop(0, NSLICE // 16, dis_step, 0)
        pltpu.sync_copy(zbuf, sp_deg.at[nsl])
        plsc.subcore_barrier()
        pltpu.sync_copy(sp_deg, dis_v)

        scope_edge = jax.named_scope("ph_edge")
        scope_edge.__enter__()

        def make_scale(j, b):
            @plsc.parallel_loop(0, 8)
            def scale_grp(gq):
                nv = nrmg[j, pl.ds(gq * 16, 16)]
                for l in range(16):
                    r = gq * 16 + l
                    s = jnp.full((16,), nv[l], jnp.float32)
                    for kk in range(8):
                        sl = pl.ds(kk * 16, 16)
                        xbuf[b, r, sl] = xbuf[b, r, sl] * s

        def edge_grp(gp, c):
            base = wid * CH_AGG + gp * GRP
            pltpu.sync_copy(row_hbm.at[pl.ds(base, GRP)], rowg)
            pltpu.sync_copy(col_hbm.at[pl.ds(base, GRP)], colg)
            pltpu.sync_copy(w_hbm.at[pl.ds(base, GRP)], nrmg)

            @plsc.parallel_loop(0, GRP)
            def nrm_step(j):
                for kk in range(8):
                    sl = pl.ds(kk * 16, 16)
                    dr = plsc.load_gather(dis_v, [rowg[j, sl]])
                    dc = plsc.load_gather(dis_v, [colg[j, sl]])
                    nrmg[j, sl] = dr * nrmg[j, sl] * dc

            cps_g = [None] * GRP
            cps_s = [None] * GRP
            cps_g[0] = pltpu.async_copy(x_hbm.at[rowg.at[0]], xbuf.at[0],
                                        sem_g[0])
            for j in range(GRP):
                b = j % 2
                nb = (j + 1) % 2
                if j + 1 < GRP:
                    if j >= 1:
                        cps_s[j - 1].wait()
                    cps_g[j + 1] = pltpu.async_copy(
                        x_hbm.at[rowg.at[j + 1]], xbuf.at[nb], sem_g[nb])
                cps_g[j].wait()
                make_scale(j, b)
                cps_s[j] = pltpu.async_copy(
                    xbuf.at[b], sp_agg.at[colg.at[j]], sem_s[b], add=True)
            cps_s[GRP - 2].wait()
            cps_s[GRP - 1].wait()
            return c
        lax.fori_loop(0, CH_AGG // GRP, edge_grp, 0)
        plsc.subcore_barrier()
        scope_edge.__exit__(None, None, None)

        pltpu.sync_copy(sp_agg.at[nsl], out_hbm.at[cid, nsl])

    return k(row2d, col2d, w2d, x_pad)


def _tc_dense(parts, h, Mz, Mr, Mh, Bz, Br, Bh, bz2, br2, bh2, linWT, linb2):
    BN = 1000
    grid = (N // BN,)

    def body(p_ref, h_ref, mz, mr, mh, bzz, brr, bhh, vz, vr, vh, lw, lb,
             y_ref, hn_ref):
        f32 = jnp.float32
        dot = functools.partial(jnp.dot, preferred_element_type=f32)
        agg = p_ref[0] + p_ref[1]
        hh = h_ref[...]
        z = jax.nn.sigmoid(dot(agg, mz[...]) + dot(hh, bzz[...]) + vz[...])
        r = jax.nn.sigmoid(dot(agg, mr[...]) + dot(hh, brr[...]) + vr[...])
        ht = jnp.tanh(dot(agg, mh[...]) + dot(hh * r, bhh[...]) + vh[...])
        hn = z * hh + (1.0 - z) * ht
        hn_ref[...] = hn
        y_ref[...] = dot(jnp.maximum(hn, 0.0), lw[...]) + lb[...]

    full = lambda shape: pl.BlockSpec(shape, lambda i: tuple(0 for _ in shape))
    return pl.pallas_call(
        body,
        grid=grid,
        in_specs=[
            pl.BlockSpec((2, BN, F), lambda i: (0, i, 0)),
            pl.BlockSpec((BN, FO), lambda i: (i, 0)),
            full((F, FO)), full((F, FO)), full((F, FO)),
            full((FO, FO)), full((FO, FO)), full((FO, FO)),
            full((1, FO)), full((1, FO)), full((1, FO)),
            full((FO, F)), full((1, F)),
        ],
        out_specs=[
            pl.BlockSpec((BN, F), lambda i: (i, 0)),
            pl.BlockSpec((BN, FO), lambda i: (i, 0)),
        ],
        out_shape=[
            jax.ShapeDtypeStruct((N, F), jnp.float32),
            jax.ShapeDtypeStruct((N, FO), jnp.float32),
        ],
    )(parts, h, Mz, Mr, Mh, Bz, Br, Bh, bz2, br2, bh2, linWT, linb2)


def kernel(g, node_feat, edge_weight, hidden_state,
           Wz, bz, Wr, br, Wh, bh, lzW, lzb, lrW, lrb, lhW, lhb, linW, linb):
    row, col = g[0], g[1]
    loop = jnp.arange(N, dtype=jnp.int32)
    padi = (jnp.arange(E_PAD - E_AUG, dtype=jnp.int32) * 37) % N
    row2d = jnp.concatenate([row, loop, padi]).reshape(EROWS, 128)
    col2d = jnp.concatenate([col, loop, padi]).reshape(EROWS, 128)
    w2d = jnp.concatenate(
        [edge_weight, jnp.ones((N,), jnp.float32),
         jnp.zeros((E_PAD - E_AUG,), jnp.float32)]).reshape(EROWS, 128)
    x_pad = jnp.pad(node_feat, ((0, N_PAD - N), (0, 0)))

    parts = _sc_aggregate(row2d, col2d, w2d, x_pad)

    Az, Bz_ = lzW[:, :FO].T, lzW[:, FO:].T
    Ar, Br_ = lrW[:, :FO].T, lrW[:, FO:].T
    Ah, Bh_ = lhW[:, :FO].T, lhW[:, FO:].T
    Mz, bz2 = Wz @ Az, (bz @ Az + lzb).reshape(1, FO)
    Mr, br2 = Wr @ Ar, (br @ Ar + lrb).reshape(1, FO)
    Mh, bh2 = Wh @ Ah, (bh @ Ah + lhb).reshape(1, FO)

    y, hn = _tc_dense(parts[:, :N], hidden_state, Mz, Mr, Mh, Bz_, Br_, Bh_,
                      bz2, br2, bh2, linW.T, linb.reshape(1, F))
    return (y, hn)


# speedup vs baseline: 1.0646x; 1.0646x over previous
import functools

import jax
import jax.numpy as jnp
from jax import lax
from jax.experimental import pallas as pl
from jax.experimental.pallas import tpu as pltpu
from jax.experimental.pallas import tpu_sc as plsc

N = 10000
E = 320000
F = 128
FO = 256

N_PAD = 10240
E_AUG = E + N
E_PAD = 360448
EROWS = E_PAD // 128
CH_AGG = EROWS // 32
CH_DEG = EROWS // 16
GRP = 8
NSLICE = N_PAD // 16


def _sc_aggregate(row2d, col2d, w2d, x_pad):
    mesh = plsc.VectorSubcoreMesh(core_axis_name="c", subcore_axis_name="s")

    @functools.partial(
        pl.kernel,
        out_type=jax.ShapeDtypeStruct((2, N_PAD, F), jnp.float32),
        mesh=mesh,
        compiler_params=pltpu.CompilerParams(needs_layout_passes=False),
        scratch_types=[
            pltpu.VMEM((GRP, 128), jnp.int32),
            pltpu.VMEM((GRP, 128), jnp.int32),
            pltpu.VMEM((GRP, 128), jnp.float32),
            pltpu.VMEM((N_PAD,), jnp.float32),
            pltpu.VMEM((2, 128, F), jnp.float32),
            pltpu.VMEM((NSLICE,), jnp.float32),
            pltpu.SemaphoreType.DMA,
            pltpu.SemaphoreType.DMA,
            pltpu.SemaphoreType.DMA,
            pltpu.SemaphoreType.DMA,
            pltpu.SemaphoreType.DMA,
            pltpu.VMEM_SHARED((N_PAD,), jnp.float32),
            pltpu.VMEM_SHARED((N_PAD, F), jnp.float32),
        ],
    )
    def k(row_hbm, col_hbm, w_hbm, x_hbm, out_hbm,
          rowg, colg, nrmg, dis_v, xbuf, zbuf, sem, sg0, sg1, ss0, ss1,
          sp_deg, sp_agg):
        sem_g = (sg0, sg1)
        sem_s = (ss0, ss1)
        cid = lax.axis_index("c")
        sid = lax.axis_index("s")
        wid = sid * 2 + cid
        nbase = sid * NSLICE
        z16 = jnp.zeros((16,), jnp.float32)
        nsl = pl.ds(nbase, NSLICE)

        def zero_xrow(i, c):
            for kk in range(8):
                xbuf[0, i, pl.ds(kk * 16, 16)] = z16
            return c
        lax.fori_loop(0, 128, zero_xrow, 0)
        for i in range(NSLICE // 16):
            zbuf[pl.ds(i * 16, 16)] = z16
        pltpu.sync_copy(zbuf, sp_deg.at[nsl])
        for j in range(NSLICE // 128):
            pltpu.sync_copy(xbuf.at[0], sp_agg.at[pl.ds(nbase + j * 128, 128)])
        plsc.subcore_barrier()

        scope_deg = jax.named_scope("ph_deg")
        scope_deg.__enter__()

        def deg_grp(gp, c):
            base = sid * CH_DEG + gp * GRP
            st = [pltpu.async_copy(col_hbm.at[pl.ds(base, GRP)], colg, sem),
                  pltpu.async_copy(w_hbm.at[pl.ds(base, GRP)], nrmg, sem)]
            for d in st:
                d.wait()
            descs = [pltpu.async_copy(nrmg.at[j], sp_deg.at[colg.at[j]], sem,
                                      add=True)
                     for j in range(GRP)]
            for d in descs:
                d.wait()
            return c
        lax.fori_loop(0, CH_DEG // GRP, deg_grp, 0)
        plsc.subcore_barrier()
        scope_deg.__exit__(None, None, None)

        pltpu.sync_copy(sp_deg.at[nsl], zbuf)

        def dis_step(i, c):
            d = zbuf[pl.ds(i * 16, 16)]
            bits = lax.bitcast_convert_type(d, jnp.int32)
            bits = jnp.int32(0x5F3759DF) - jnp.right_shift(bits, jnp.int32(1))
            y = lax.bitcast_convert_type(bits, jnp.float32)
            half = -0.5 * d
            for _ in range(3):
                y = y * (1.5 + half * y * y)
            zbuf[pl.ds(i * 16, 16)] = y
            return c
        lax.fori_loop(0, NSLICE // 16, dis_step, 0)
        pltpu.sync_copy(zbuf, sp_deg.at[nsl])
        plsc.subcore_barrier()
        pltpu.sync_copy(sp_deg, dis_v)

        scope_edge = jax.named_scope("ph_edge")
        scope_edge.__enter__()

        def make_scale(j, b):
            @plsc.parallel_loop(0, 8)
            def scale_grp(gq):
                nv = nrmg[j, pl.ds(gq * 16, 16)]
                for l in range(16):
                    r = gq * 16 + l
                    s = jnp.full((16,), nv[l], jnp.float32)
                    for kk in range(8):
                        sl = pl.ds(kk * 16, 16)
                        xbuf[b, r, sl] = xbuf[b, r, sl] * s

        def edge_grp(gp, c):
            base = wid * CH_AGG + gp * GRP
            st = [pltpu.async_copy(row_hbm.at[pl.ds(base, GRP)], rowg, sem),
                  pltpu.async_copy(col_hbm.at[pl.ds(base, GRP)], colg, sem),
                  pltpu.async_copy(w_hbm.at[pl.ds(base, GRP)], nrmg, sem)]
            for d in st:
                d.wait()

            @plsc.parallel_loop(0, GRP)
            def nrm_step(j):
                for kk in range(8):
                    sl = pl.ds(kk * 16, 16)
                    dr = plsc.load_gather(dis_v, [rowg[j, sl]])
                    dc = plsc.load_gather(dis_v, [colg[j, sl]])
                    nrmg[j, sl] = dr * nrmg[j, sl] * dc

            cps_g = [None] * GRP
            cps_s = [None] * GRP
            cps_g[0] = pltpu.async_copy(x_hbm.at[rowg.at[0]], xbuf.at[0],
                                        sem_g[0])
            for j in range(GRP):
                b = j % 2
                nb = (j + 1) % 2
                if j + 1 < GRP:
                    if j >= 1:
                        cps_s[j - 1].wait()
                    cps_g[j + 1] = pltpu.async_copy(
                        x_hbm.at[rowg.at[j + 1]], xbuf.at[nb], sem_g[nb])
                cps_g[j].wait()
                make_scale(j, b)
                cps_s[j] = pltpu.async_copy(
                    xbuf.at[b], sp_agg.at[colg.at[j]], sem_s[b], add=True)
            cps_s[GRP - 2].wait()
            cps_s[GRP - 1].wait()
            return c
        lax.fori_loop(0, CH_AGG // GRP, edge_grp, 0)
        plsc.subcore_barrier()
        scope_edge.__exit__(None, None, None)

        pltpu.sync_copy(sp_agg.at[nsl], out_hbm.at[cid, nsl])

    return k(row2d, col2d, w2d, x_pad)


def _tc_dense(parts, h, Mz, Mr, Mh, Bz, Br, Bh, bz2, br2, bh2, linWT, linb2):
    BN = 1000
    grid = (N // BN,)

    def body(p_ref, h_ref, mz, mr, mh, bzz, brr, bhh, vz, vr, vh, lw, lb,
             y_ref, hn_ref):
        f32 = jnp.float32
        dot = functools.partial(jnp.dot, preferred_element_type=f32)
        agg = p_ref[0] + p_ref[1]
        hh = h_ref[...]
        z = jax.nn.sigmoid(dot(agg, mz[...]) + dot(hh, bzz[...]) + vz[...])
        r = jax.nn.sigmoid(dot(agg, mr[...]) + dot(hh, brr[...]) + vr[...])
        ht = jnp.tanh(dot(agg, mh[...]) + dot(hh * r, bhh[...]) + vh[...])
        hn = z * hh + (1.0 - z) * ht
        hn_ref[...] = hn
        y_ref[...] = dot(jnp.maximum(hn, 0.0), lw[...]) + lb[...]

    full = lambda shape: pl.BlockSpec(shape, lambda i: tuple(0 for _ in shape))
    return pl.pallas_call(
        body,
        grid=grid,
        in_specs=[
            pl.BlockSpec((2, BN, F), lambda i: (0, i, 0)),
            pl.BlockSpec((BN, FO), lambda i: (i, 0)),
            full((F, FO)), full((F, FO)), full((F, FO)),
            full((FO, FO)), full((FO, FO)), full((FO, FO)),
            full((1, FO)), full((1, FO)), full((1, FO)),
            full((FO, F)), full((1, F)),
        ],
        out_specs=[
            pl.BlockSpec((BN, F), lambda i: (i, 0)),
            pl.BlockSpec((BN, FO), lambda i: (i, 0)),
        ],
        out_shape=[
            jax.ShapeDtypeStruct((N, F), jnp.float32),
            jax.ShapeDtypeStruct((N, FO), jnp.float32),
        ],
    )(parts, h, Mz, Mr, Mh, Bz, Br, Bh, bz2, br2, bh2, linWT, linb2)


def kernel(g, node_feat, edge_weight, hidden_state,
           Wz, bz, Wr, br, Wh, bh, lzW, lzb, lrW, lrb, lhW, lhb, linW, linb):
    row, col = g[0], g[1]
    loop = jnp.arange(N, dtype=jnp.int32)
    padi = (jnp.arange(E_PAD - E_AUG, dtype=jnp.int32) * 37) % N
    row2d = jnp.concatenate([row, loop, padi]).reshape(EROWS, 128)
    col2d = jnp.concatenate([col, loop, padi]).reshape(EROWS, 128)
    w2d = jnp.concatenate(
        [edge_weight, jnp.ones((N,), jnp.float32),
         jnp.zeros((E_PAD - E_AUG,), jnp.float32)]).reshape(EROWS, 128)
    x_pad = jnp.pad(node_feat, ((0, N_PAD - N), (0, 0)))

    parts = _sc_aggregate(row2d, col2d, w2d, x_pad)

    Az, Bz_ = lzW[:, :FO].T, lzW[:, FO:].T
    Ar, Br_ = lrW[:, :FO].T, lrW[:, FO:].T
    Ah, Bh_ = lhW[:, :FO].T, lhW[:, FO:].T
    Mz, bz2 = Wz @ Az, (bz @ Az + lzb).reshape(1, FO)
    Mr, br2 = Wr @ Ar, (br @ Ar + lrb).reshape(1, FO)
    Mh, bh2 = Wh @ Ah, (bh @ Ah + lhb).reshape(1, FO)

    y, hn = _tc_dense(parts[:, :N], hidden_state, Mz, Mr, Mh, Bz_, Br_, Bh_,
                      bz2, br2, bh2, linW.T, linb.reshape(1, F))
    return (y, hn)
